# Initial kernel scaffold; baseline (speedup 1.0000x reference)
import functools

import jax
import jax.numpy as jnp
from jax import lax
from jax.experimental import pallas as pl
from jax.experimental.pallas import tpu as pltpu
from jax.experimental.pallas import tpu_sc as plsc

N = 10000
E = 320000
D = 128

NC = 2
NS = 16
NW = NC * NS
EPW = E // NW
K = 80
CH = EPW // K
RPT = N // NS
ZR = 25


def _scatter_body(h_hbm, src_hbm, dst_hbm, out_hbm,
                  sidx, didx, rows, zbuf, acc, sem):
    c = lax.axis_index("c")
    s = lax.axis_index("s")
    wid = c * NS + s

    for r in range(ZR):
        for q in range(D // 16):
            zbuf[r, pl.ds(q * 16, 16)] = jnp.zeros((16,), jnp.float32)
    row0 = s * RPT
    for t in range(RPT // ZR):
        pltpu.sync_copy(zbuf, acc.at[pl.ds(row0 + t * ZR, ZR)])
    plsc.subcore_barrier()

    base = wid * EPW

    def chunk(k, carry):
        off = base + k * K
        pltpu.sync_copy(src_hbm.at[pl.ds(off, K)], sidx)
        pltpu.sync_copy(dst_hbm.at[pl.ds(off, K)], didx)
        pltpu.async_copy(h_hbm.at[sidx], rows, sem).wait()
        pltpu.sync_copy(rows, acc.at[didx], add=True)
        return carry

    lax.fori_loop(0, CH, chunk, 0)
    plsc.subcore_barrier()

    pltpu.sync_copy(acc.at[pl.ds(row0, RPT)],
                    out_hbm.at[c, pl.ds(row0, RPT)])


@jax.jit
def _scatter_partials(h, src, dst):
    mesh = plsc.VectorSubcoreMesh(core_axis_name="c", subcore_axis_name="s")
    f = pl.kernel(
        _scatter_body,
        out_type=jax.ShapeDtypeStruct((NC, N, D), jnp.float32),
        mesh=mesh,
        scratch_types=[
            pltpu.VMEM((K,), jnp.int32),
            pltpu.VMEM((K,), jnp.int32),
            pltpu.VMEM((K, D), jnp.float32),
            pltpu.VMEM((ZR, D), jnp.float32),
            pltpu.VMEM_SHARED((N, D), jnp.float32),
            pltpu.SemaphoreType.DMA,
        ],
    )
    return f(h, src, dst)


BN = 2000


def _mm_body(x_ref, w_ref, b_ref, o_ref, *, relu):
    acc = lax.dot_general(x_ref[...], w_ref[...],
                          dimension_numbers=(((1,), (1,)), ((), ())),
                          preferred_element_type=jnp.float32,
                          precision=lax.Precision.HIGHEST)
    acc = acc + b_ref[...]
    o_ref[...] = jnp.maximum(acc, 0.0) if relu else acc


def _mm_agg_body(x_ref, p0_ref, p1_ref, w_ref, b_ref, o_ref, *, relu):
    hh = x_ref[...] + p0_ref[...] + p1_ref[...]
    acc = lax.dot_general(hh, w_ref[...],
                          dimension_numbers=(((1,), (1,)), ((), ())),
                          preferred_element_type=jnp.float32,
                          precision=lax.Precision.HIGHEST)
    acc = acc + b_ref[...]
    o_ref[...] = jnp.maximum(acc, 0.0) if relu else acc


_row_spec = pl.BlockSpec((BN, D), lambda i: (i, 0))
_full_spec = pl.BlockSpec((D, D), lambda i: (0, 0))
_b_spec = pl.BlockSpec((1, D), lambda i: (0, 0))


def _linear(x, w, b, relu=False):
    return pl.pallas_call(
        functools.partial(_mm_body, relu=relu),
        grid=(N // BN,),
        in_specs=[_row_spec, _full_spec, _b_spec],
        out_specs=_row_spec,
        out_shape=jax.ShapeDtypeStruct((N, D), jnp.float32),
    )(x, w, b.reshape(1, D))


def _linear_agg(x, p0, p1, w, b, relu=False):
    return pl.pallas_call(
        functools.partial(_mm_agg_body, relu=relu),
        grid=(N // BN,),
        in_specs=[_row_spec, _row_spec, _row_spec, _full_spec, _b_spec],
        out_specs=_row_spec,
        out_shape=jax.ShapeDtypeStruct((N, D), jnp.float32),
    )(x, w, b.reshape(1, D))


def kernel(x, edge_index, W_pre, b_pre, W1, b1, W2, b2):
    src = edge_index[0]
    dst = edge_index[1]
    h0 = _linear(x, W_pre, b_pre)
    p = _scatter_partials(h0, src, dst)
    h1 = _linear_agg(h0, p[0], p[1], W1, b1, relu=True)
    q = _scatter_partials(h1, src, dst)
    return _linear_agg(h1, q[0], q[1], W2, b2, relu=False)


# speedup vs baseline: 4.5587x; 4.5587x over previous
import functools

import jax
import jax.numpy as jnp
from jax import lax
from jax.experimental import pallas as pl
from jax.experimental.pallas import tpu as pltpu
from jax.experimental.pallas import tpu_sc as plsc

N = 10000
E = 320000
D = 128

NC = 2
NS = 16
NW = NC * NS
EPW = E // NW
K = 80
CH = EPW // K
RPT = 624
TAIL = N - NS * RPT
ZR = 24


def _scatter_body(h_hbm, src_hbm, dst_hbm, out_hbm,
                  sidx, didx, rows, zbuf, acc, sem):
    c = lax.axis_index("c")
    s = lax.axis_index("s")
    wid = c * NS + s

    for r in range(ZR):
        for q in range(D // 16):
            zbuf[r, pl.ds(q * 16, 16)] = jnp.zeros((16,), jnp.float32)
    row0 = s * RPT
    for t in range(RPT // ZR):
        pltpu.sync_copy(zbuf, acc.at[pl.ds(row0 + t * ZR, ZR)])

    @pl.when(s == 0)
    def _zero_tail():
        pltpu.sync_copy(zbuf.at[pl.ds(0, TAIL)], acc.at[pl.ds(NS * RPT, TAIL)])

    plsc.subcore_barrier()

    base = wid * EPW

    def chunk(k, carry):
        off = base + k * K
        pltpu.sync_copy(src_hbm.at[pl.ds(off, K)], sidx)
        pltpu.sync_copy(dst_hbm.at[pl.ds(off, K)], didx)
        pltpu.async_copy(h_hbm.at[sidx], rows, sem).wait()
        pltpu.sync_copy(rows, acc.at[didx], add=True)
        return carry

    lax.fori_loop(0, CH, chunk, 0)
    plsc.subcore_barrier()

    pltpu.sync_copy(acc.at[pl.ds(row0, RPT)],
                    out_hbm.at[c, pl.ds(row0, RPT)])

    @pl.when(s == 0)
    def _write_tail():
        pltpu.sync_copy(acc.at[pl.ds(NS * RPT, TAIL)],
                        out_hbm.at[c, pl.ds(NS * RPT, TAIL)])


@jax.jit
def _scatter_partials(h, src, dst):
    mesh = plsc.VectorSubcoreMesh(core_axis_name="c", subcore_axis_name="s")
    f = pl.kernel(
        _scatter_body,
        out_type=jax.ShapeDtypeStruct((NC, N, D), jnp.float32),
        mesh=mesh,
        scratch_types=[
            pltpu.VMEM((K,), jnp.int32),
            pltpu.VMEM((K,), jnp.int32),
            pltpu.VMEM((K, D), jnp.float32),
            pltpu.VMEM((ZR, D), jnp.float32),
            pltpu.VMEM_SHARED((N, D), jnp.float32),
            pltpu.SemaphoreType.DMA,
        ],
    )
    return f(h, src, dst)


BN = 2000


def _mm_body(x_ref, w_ref, b_ref, o_ref, *, relu):
    acc = lax.dot_general(x_ref[...], w_ref[...],
                          dimension_numbers=(((1,), (1,)), ((), ())),
                          preferred_element_type=jnp.float32,
                          precision=lax.Precision.HIGHEST)
    acc = acc + b_ref[...]
    o_ref[...] = jnp.maximum(acc, 0.0) if relu else acc


def _mm_agg_body(x_ref, p0_ref, p1_ref, w_ref, b_ref, o_ref, *, relu):
    hh = x_ref[...] + p0_ref[...] + p1_ref[...]
    acc = lax.dot_general(hh, w_ref[...],
                          dimension_numbers=(((1,), (1,)), ((), ())),
                          preferred_element_type=jnp.float32,
                          precision=lax.Precision.HIGHEST)
    acc = acc + b_ref[...]
    o_ref[...] = jnp.maximum(acc, 0.0) if relu else acc


_row_spec = pl.BlockSpec((BN, D), lambda i: (i, 0))
_full_spec = pl.BlockSpec((D, D), lambda i: (0, 0))
_b_spec = pl.BlockSpec((1, D), lambda i: (0, 0))


def _linear(x, w, b, relu=False):
    return pl.pallas_call(
        functools.partial(_mm_body, relu=relu),
        grid=(N // BN,),
        in_specs=[_row_spec, _full_spec, _b_spec],
        out_specs=_row_spec,
        out_shape=jax.ShapeDtypeStruct((N, D), jnp.float32),
    )(x, w, b.reshape(1, D))


def _linear_agg(x, p0, p1, w, b, relu=False):
    return pl.pallas_call(
        functools.partial(_mm_agg_body, relu=relu),
        grid=(N // BN,),
        in_specs=[_row_spec, _row_spec, _row_spec, _full_spec, _b_spec],
        out_specs=_row_spec,
        out_shape=jax.ShapeDtypeStruct((N, D), jnp.float32),
    )(x, p0, p1, w, b.reshape(1, D))


def kernel(x, edge_index, W_pre, b_pre, W1, b1, W2, b2):
    src = edge_index[0]
    dst = edge_index[1]
    h0 = _linear(x, W_pre, b_pre)
    p = _scatter_partials(h0, src, dst)
    h1 = _linear_agg(h0, p[0], p[1], W1, b1, relu=True)
    q = _scatter_partials(h1, src, dst)
    return _linear_agg(h1, q[0], q[1], W2, b2, relu=False)


# speedup vs baseline: 10.1091x; 2.2175x over previous
import functools

import jax
import jax.numpy as jnp
from jax import lax
from jax.experimental import pallas as pl
from jax.experimental.pallas import tpu as pltpu
from jax.experimental.pallas import tpu_sc as plsc

N = 10000
E = 320000
D = 128

NC = 2
NS = 16
NW = NC * NS
EPW = E // NW
K = 80
CH = EPW // K
RPT = 624
TAIL = N - NS * RPT
ZR = 24


def _scatter_body(h_hbm, pk_hbm, out_hbm,
                  pk, sb0, db0, sb1, db1, rows0, rows1, zbuf, acc,
                  sem0, sem1):
    c = lax.axis_index("c")
    s = lax.axis_index("s")
    wid = c * NS + s

    pltpu.sync_copy(pk_hbm.at[wid], pk)

    for r in range(ZR):
        for q in range(D // 16):
            zbuf[r, pl.ds(q * 16, 16)] = jnp.zeros((16,), jnp.float32)
    row0 = s * RPT
    for t in range(RPT // ZR):
        pltpu.sync_copy(zbuf, acc.at[pl.ds(row0 + t * ZR, ZR)])

    @pl.when(s == 0)
    def _zero_tail():
        pltpu.sync_copy(zbuf.at[pl.ds(0, TAIL)], acc.at[pl.ds(NS * RPT, TAIL)])

    plsc.subcore_barrier()

    def unpack(k, sb, db):
        for q in range(K // 16):
            v = pk[k, pl.ds(q * 16, 16)]
            sb[pl.ds(q * 16, 16)] = v >> 14
            db[pl.ds(q * 16, 16)] = v & 16383

    def gather(sb, buf, sem):
        pltpu.async_copy(h_hbm.at[sb], buf, sem)

    def gwait(sb, buf, sem):
        pltpu.make_async_copy(h_hbm.at[sb], buf, sem).wait()

    def scat(buf, db):
        pltpu.sync_copy(buf, acc.at[db], add=True)

    unpack(0, sb0, db0)
    gather(sb0, rows0, sem0)

    def pipe(j, carry):
        k0 = 2 * j
        unpack(k0 + 1, sb1, db1)
        gather(sb1, rows1, sem1)
        gwait(sb0, rows0, sem0)
        scat(rows0, db0)
        unpack(k0 + 2, sb0, db0)
        gather(sb0, rows0, sem0)
        gwait(sb1, rows1, sem1)
        scat(rows1, db1)
        return carry

    lax.fori_loop(0, (CH - 1) // 2, pipe, 0)
    gwait(sb0, rows0, sem0)
    scat(rows0, db0)
    plsc.subcore_barrier()

    pltpu.sync_copy(acc.at[pl.ds(row0, RPT)],
                    out_hbm.at[c, pl.ds(row0, RPT)])

    @pl.when(s == 0)
    def _write_tail():
        pltpu.sync_copy(acc.at[pl.ds(NS * RPT, TAIL)],
                        out_hbm.at[c, pl.ds(NS * RPT, TAIL)])


@jax.jit
def _scatter_partials(h, packed):
    mesh = plsc.VectorSubcoreMesh(core_axis_name="c", subcore_axis_name="s")
    f = pl.kernel(
        _scatter_body,
        out_type=jax.ShapeDtypeStruct((NC, N, D), jnp.float32),
        mesh=mesh,
        scratch_types=[
            pltpu.VMEM((CH, K), jnp.int32),
            pltpu.VMEM((K,), jnp.int32),
            pltpu.VMEM((K,), jnp.int32),
            pltpu.VMEM((K,), jnp.int32),
            pltpu.VMEM((K,), jnp.int32),
            pltpu.VMEM((K, D), jnp.float32),
            pltpu.VMEM((K, D), jnp.float32),
            pltpu.VMEM((ZR, D), jnp.float32),
            pltpu.VMEM_SHARED((N, D), jnp.float32),
            pltpu.SemaphoreType.DMA,
            pltpu.SemaphoreType.DMA,
        ],
    )
    return f(h, packed)


BN = 2000


def _mm_body(x_ref, w_ref, b_ref, o_ref, *, relu):
    acc = lax.dot_general(x_ref[...], w_ref[...],
                          dimension_numbers=(((1,), (1,)), ((), ())),
                          preferred_element_type=jnp.float32,
                          precision=lax.Precision.HIGHEST)
    acc = acc + b_ref[...]
    o_ref[...] = jnp.maximum(acc, 0.0) if relu else acc


def _mm_agg_body(x_ref, p0_ref, p1_ref, w_ref, b_ref, o_ref, *, relu):
    hh = x_ref[...] + p0_ref[...] + p1_ref[...]
    acc = lax.dot_general(hh, w_ref[...],
                          dimension_numbers=(((1,), (1,)), ((), ())),
                          preferred_element_type=jnp.float32,
                          precision=lax.Precision.HIGHEST)
    acc = acc + b_ref[...]
    o_ref[...] = jnp.maximum(acc, 0.0) if relu else acc


_row_spec = pl.BlockSpec((BN, D), lambda i: (i, 0))
_full_spec = pl.BlockSpec((D, D), lambda i: (0, 0))
_b_spec = pl.BlockSpec((1, D), lambda i: (0, 0))


def _linear(x, w, b, relu=False):
    return pl.pallas_call(
        functools.partial(_mm_body, relu=relu),
        grid=(N // BN,),
        in_specs=[_row_spec, _full_spec, _b_spec],
        out_specs=_row_spec,
        out_shape=jax.ShapeDtypeStruct((N, D), jnp.float32),
    )(x, w, b.reshape(1, D))


def _linear_agg(x, p0, p1, w, b, relu=False):
    return pl.pallas_call(
        functools.partial(_mm_agg_body, relu=relu),
        grid=(N // BN,),
        in_specs=[_row_spec, _row_spec, _row_spec, _full_spec, _b_spec],
        out_specs=_row_spec,
        out_shape=jax.ShapeDtypeStruct((N, D), jnp.float32),
    )(x, p0, p1, w, b.reshape(1, D))


def kernel(x, edge_index, W_pre, b_pre, W1, b1, W2, b2):
    packed = ((edge_index[0] << 14) | edge_index[1]).reshape(NW, CH, K)
    h0 = _linear(x, W_pre, b_pre)
    p = _scatter_partials(h0, packed)
    h1 = _linear_agg(h0, p[0], p[1], W1, b1, relu=True)
    q = _scatter_partials(h1, packed)
    return _linear_agg(h1, q[0], q[1], W2, b2, relu=False)
